# baseline (speedup 1.0000x reference)
import functools

import jax
import jax.numpy as jnp
from jax import lax
from jax.experimental import pallas as pl
from jax.experimental.pallas import tpu as pltpu
from jax.experimental.pallas import tpu_sc as plsc

TSCALE = 2048
C = 128
BS = 16
N = 2048
T = TSCALE
NTAP = 6

NC = 2
NS = 16
L = 16
NW = NC * NS

PPT = (BS * N) // NW
K = 128
CPB = K // L
NBLK = PPT // K


_TT = 2048


def _proj_body(feat_ref, w_ref, b_ref, out_ref):
    for b in range(4):
        f = feat_ref[b].astype(jnp.bfloat16)
        for k in range(NTAP):
            acc = lax.dot_general(f, w_ref[k], (((0,), (0,)), ((), ())),
                                  preferred_element_type=jnp.float32)
            out_ref[k, b] = acc + b_ref[k]


def _project(feat_t, wstack, bias):
    grid = (BS // 4, T // _TT)
    return pl.pallas_call(
        _proj_body,
        grid=grid,
        in_specs=[
            pl.BlockSpec((4, C, _TT), lambda b, t: (b, 0, t)),
            pl.BlockSpec((NTAP, C, C), lambda b, t: (0, 0, 0)),
            pl.BlockSpec((NTAP, 1, C), lambda b, t: (0, 0, 0)),
        ],
        out_specs=pl.BlockSpec((NTAP, 4, _TT, C), lambda b, t: (0, b, t, 0)),
        out_shape=jax.ShapeDtypeStruct((NTAP, BS, T, C), jnp.float32),
        compiler_params=pltpu.CompilerParams(
            fuse_transposed_lhs_in_matmul=True),
    )(feat_t, wstack, bias)


@functools.cache
def _build_sc_combine():
    mesh = plsc.VectorSubcoreMesh(core_axis_name="c", subcore_axis_name="s",
                                  num_cores=NC, num_subcores=NS)
    return pl.kernel(
        _sc_combine_body,
        out_type=[
            jax.ShapeDtypeStruct((BS * N,), jnp.float32),
            jax.ShapeDtypeStruct((BS * N,), jnp.float32),
        ],
        mesh=mesh,
        scratch_types=[
            pltpu.VMEM((PPT,), jnp.float32),
            pltpu.VMEM((PPT,), jnp.float32),
            pltpu.VMEM((NBLK * NTAP, K), jnp.int32),
            pltpu.VMEM((2, 3, K, C), jnp.float32),
            pltpu.VMEM((2, 160), jnp.float32),
            pltpu.VMEM((2 * PPT,), jnp.float32),
            pltpu.VMEM((L, L), jnp.float32),
            pltpu.SemaphoreType.DMA,
            pltpu.SemaphoreType.DMA,
        ],
        compiler_params=pltpu.CompilerParams(needs_layout_passes=False),
    )


def _sc_combine_body(tab_hbm, locs_hbm, loce_hbm, params_hbm,
                     outs_hbm, oute_hbm,
                     locs_v, loce_v, idx_v, g_v, par_v, o_v, ts_v,
                     sem0, sem1):
    sems = (sem0, sem1)
    wid = lax.axis_index("s") * NC + lax.axis_index("c")
    base = wid * PPT
    b_off = (wid // (N // PPT)) * T
    rows16 = lax.iota(jnp.int32, L)

    pltpu.sync_copy(locs_hbm.at[pl.ds(base, PPT)], locs_v)
    pltpu.sync_copy(loce_hbm.at[pl.ds(base, PPT)], loce_v)
    pltpu.sync_copy(params_hbm, par_v)

    def idx_body(i, carry):
        ls = jnp.clip(locs_v[pl.ds(i * L, L)], 0.0, float(TSCALE - 1))
        le = jnp.clip(loce_v[pl.ds(i * L, L)], 0.0, float(TSCALE - 1))
        blen = (le - ls + 1.0) * 0.125

        def to_idx(v):
            return jnp.clip(v.astype(jnp.int32), 0, TSCALE - 1)

        vals = (to_idx(ls - blen), to_idx(ls), to_idx(ls + blen),
                to_idx(le - blen), to_idx(le), to_idx(le + blen))
        g = i // CPB
        cc = i % CPB
        for k in range(NTAP):
            idx_v[g * NTAP + k, pl.ds(cc * L, L)] = (
                vals[k] + (k * BS * T + b_off))
        return carry

    lax.fori_loop(0, PPT // L, idx_body, 0)

    def issue(u, buf):
        row0 = (u >> 1) * NTAP + (u & 1) * 3
        for k in range(3):
            pltpu.async_copy(tab_hbm.at[idx_v.at[row0 + k]],
                             g_v.at[buf, k], sems[buf])

    def drain(u, buf):
        row0 = (u >> 1) * NTAP + (u & 1) * 3
        for k in range(3):
            pltpu.make_async_copy(tab_hbm.at[idx_v.at[row0 + k]],
                                  g_v.at[buf, k], sems[buf]).wait()

    def compute(u, buf):
        br = u & 1
        obase = br * PPT + (u >> 1) * K
        w2 = [par_v[br, pl.ds(cc * L, L)] for cc in range(C // L)]
        b2 = par_v[br, pl.ds(C, L)]

        def cbody(c, carry):
            for jj in range(L):
                j = c * L + jj
                acc = None
                for cc in range(C // L):
                    sl = pl.ds(cc * L, L)
                    h = (g_v[buf, 0, j, sl] + g_v[buf, 1, j, sl]
                         + g_v[buf, 2, j, sl])
                    h = jnp.maximum(h, 0.0) * w2[cc]
                    acc = h if acc is None else acc + h
                ts_v[jj] = acc
            tot = b2
            for cc in range(L):
                cols = jnp.full((L,), cc, jnp.int32)
                tot = tot + plsc.load_gather(ts_v, [rows16, cols])
            o_v[pl.ds(obase + c * L, L)] = tot
            return carry

        lax.fori_loop(0, CPB, cbody, 0)

    issue(0, 0)
    issue(1, 1)
    nunit = 2 * NBLK

    def super_body(h, carry):
        for buf in range(2):
            u = 2 * h + buf
            drain(u, buf)
            compute(u, buf)

            @pl.when(u + 2 < nunit)
            def _():
                issue(u + 2, buf)
        return carry

    lax.fori_loop(0, nunit // 2, super_body, 0)

    pltpu.sync_copy(o_v.at[pl.ds(0, PPT)], outs_hbm.at[pl.ds(base, PPT)])
    pltpu.sync_copy(o_v.at[pl.ds(PPT, PPT)], oute_hbm.at[pl.ds(base, PPT)])


def kernel(loc_box, feat_frmlvl, start_w1, start_b1, start_w2, start_b2,
           end_w1, end_b1, end_w2, end_b2):
    wstack = (jnp.stack([start_w1, end_w1])
              .transpose(0, 3, 2, 1)
              .reshape(NTAP, C, C).astype(jnp.bfloat16))
    bias = jnp.zeros((NTAP, 1, C), jnp.float32)
    bias = bias.at[1, 0].set(start_b1).at[4, 0].set(end_b1)

    tab = _project(feat_frmlvl, wstack, bias).reshape(NTAP * BS * T, C)

    locs = loc_box[:, :, 0].reshape(-1)
    loce = loc_box[:, :, 1].reshape(-1)
    params = jnp.stack([
        jnp.concatenate([start_w2[0, :, 0], jnp.broadcast_to(start_b2, (32,))]),
        jnp.concatenate([end_w2[0, :, 0], jnp.broadcast_to(end_b2, (32,))]),
    ])

    outs, oute = _build_sc_combine()(tab, locs, loce, params)
    return outs.reshape(BS, N), oute.reshape(BS, N)


# speedup vs baseline: 1.0026x; 1.0026x over previous
import functools

import jax
import jax.numpy as jnp
from jax import lax
from jax.experimental import pallas as pl
from jax.experimental.pallas import tpu as pltpu
from jax.experimental.pallas import tpu_sc as plsc

TSCALE = 2048
C = 128
BS = 16
N = 2048
T = TSCALE
NTAP = 6

NC = 2
NS = 16
L = 16
NW = NC * NS

PPT = (BS * N) // NW
K = 128
CPB = K // L
NBLK = PPT // K


_TT = 2048


def _proj_body(feat_ref, w_ref, b_ref, out_ref):
    for b in range(2):
        f = feat_ref[b].astype(jnp.bfloat16)
        for k in range(NTAP):
            acc = lax.dot_general(f, w_ref[k], (((0,), (0,)), ((), ())),
                                  preferred_element_type=jnp.float32)
            out_ref[k, b] = acc + b_ref[k]


def _project(feat_t, wstack, bias):
    grid = (BS // 2, T // _TT)
    return pl.pallas_call(
        _proj_body,
        grid=grid,
        in_specs=[
            pl.BlockSpec((2, C, _TT), lambda b, t: (b, 0, t)),
            pl.BlockSpec((NTAP, C, C), lambda b, t: (0, 0, 0)),
            pl.BlockSpec((NTAP, 1, C), lambda b, t: (0, 0, 0)),
        ],
        out_specs=pl.BlockSpec((NTAP, 2, _TT, C), lambda b, t: (0, b, t, 0)),
        out_shape=jax.ShapeDtypeStruct((NTAP, BS, T, C), jnp.float32),
        compiler_params=pltpu.CompilerParams(
            fuse_transposed_lhs_in_matmul=True),
    )(feat_t, wstack, bias)


@functools.cache
def _build_sc_combine():
    mesh = plsc.VectorSubcoreMesh(core_axis_name="c", subcore_axis_name="s",
                                  num_cores=NC, num_subcores=NS)
    return pl.kernel(
        _sc_combine_body,
        out_type=[
            jax.ShapeDtypeStruct((BS * N,), jnp.float32),
            jax.ShapeDtypeStruct((BS * N,), jnp.float32),
        ],
        mesh=mesh,
        scratch_types=[
            pltpu.VMEM((PPT,), jnp.float32),
            pltpu.VMEM((PPT,), jnp.float32),
            pltpu.VMEM((NBLK * NTAP, K), jnp.int32),
            pltpu.VMEM((2, 3, K, C), jnp.float32),
            pltpu.VMEM((2, 160), jnp.float32),
            pltpu.VMEM((2 * PPT,), jnp.float32),
            pltpu.VMEM((L, L), jnp.float32),
            pltpu.SemaphoreType.DMA,
            pltpu.SemaphoreType.DMA,
        ],
        compiler_params=pltpu.CompilerParams(needs_layout_passes=False),
    )


def _sc_combine_body(tab_hbm, locs_hbm, loce_hbm, params_hbm,
                     outs_hbm, oute_hbm,
                     locs_v, loce_v, idx_v, g_v, par_v, o_v, ts_v,
                     sem0, sem1):
    sems = (sem0, sem1)
    wid = lax.axis_index("s") * NC + lax.axis_index("c")
    base = wid * PPT
    b_off = (wid // (N // PPT)) * T
    rows16 = lax.iota(jnp.int32, L)

    pltpu.sync_copy(locs_hbm.at[pl.ds(base, PPT)], locs_v)
    pltpu.sync_copy(loce_hbm.at[pl.ds(base, PPT)], loce_v)
    pltpu.sync_copy(params_hbm, par_v)

    def idx_body(i, carry):
        ls = jnp.clip(locs_v[pl.ds(i * L, L)], 0.0, float(TSCALE - 1))
        le = jnp.clip(loce_v[pl.ds(i * L, L)], 0.0, float(TSCALE - 1))
        blen = (le - ls + 1.0) * 0.125

        def to_idx(v):
            return jnp.clip(v.astype(jnp.int32), 0, TSCALE - 1)

        vals = (to_idx(ls - blen), to_idx(ls), to_idx(ls + blen),
                to_idx(le - blen), to_idx(le), to_idx(le + blen))
        g = i // CPB
        cc = i % CPB
        for k in range(NTAP):
            idx_v[g * NTAP + k, pl.ds(cc * L, L)] = (
                vals[k] + (k * BS * T + b_off))
        return carry

    lax.fori_loop(0, PPT // L, idx_body, 0)

    def issue(u, buf):
        row0 = (u >> 1) * NTAP + (u & 1) * 3
        for k in range(3):
            pltpu.async_copy(tab_hbm.at[idx_v.at[row0 + k]],
                             g_v.at[buf, k], sems[buf])

    def drain(u, buf):
        row0 = (u >> 1) * NTAP + (u & 1) * 3
        for k in range(3):
            pltpu.make_async_copy(tab_hbm.at[idx_v.at[row0 + k]],
                                  g_v.at[buf, k], sems[buf]).wait()

    def compute(u, buf):
        br = u & 1
        obase = br * PPT + (u >> 1) * K
        w2 = [par_v[br, pl.ds(cc * L, L)] for cc in range(C // L)]
        b2 = par_v[br, pl.ds(C, L)]

        def cbody(c, carry):
            for jj in range(L):
                j = c * L + jj
                acc = None
                for cc in range(C // L):
                    sl = pl.ds(cc * L, L)
                    h = (g_v[buf, 0, j, sl] + g_v[buf, 1, j, sl]
                         + g_v[buf, 2, j, sl])
                    h = jnp.maximum(h, 0.0) * w2[cc]
                    acc = h if acc is None else acc + h
                ts_v[jj] = acc
            tot = b2
            for cc in range(L):
                cols = jnp.full((L,), cc, jnp.int32)
                tot = tot + plsc.load_gather(ts_v, [rows16, cols])
            o_v[pl.ds(obase + c * L, L)] = tot
            return carry

        lax.fori_loop(0, CPB, cbody, 0)

    issue(0, 0)
    issue(1, 1)
    nunit = 2 * NBLK

    def super_body(h, carry):
        for buf in range(2):
            u = 2 * h + buf
            drain(u, buf)
            compute(u, buf)

            @pl.when(u + 2 < nunit)
            def _():
                issue(u + 2, buf)
        return carry

    lax.fori_loop(0, nunit // 2, super_body, 0)

    pltpu.sync_copy(o_v.at[pl.ds(0, PPT)], outs_hbm.at[pl.ds(base, PPT)])
    pltpu.sync_copy(o_v.at[pl.ds(PPT, PPT)], oute_hbm.at[pl.ds(base, PPT)])


def kernel(loc_box, feat_frmlvl, start_w1, start_b1, start_w2, start_b2,
           end_w1, end_b1, end_w2, end_b2):
    wstack = (jnp.stack([start_w1, end_w1])
              .transpose(0, 3, 2, 1)
              .reshape(NTAP, C, C).astype(jnp.bfloat16))
    bias = jnp.zeros((NTAP, 1, C), jnp.float32)
    bias = bias.at[1, 0].set(start_b1).at[4, 0].set(end_b1)

    tab = _project(feat_frmlvl, wstack, bias).reshape(NTAP * BS * T, C)

    locs = loc_box[:, :, 0].reshape(-1)
    loce = loc_box[:, :, 1].reshape(-1)
    params = jnp.stack([
        jnp.concatenate([start_w2[0, :, 0], jnp.broadcast_to(start_b2, (32,))]),
        jnp.concatenate([end_w2[0, :, 0], jnp.broadcast_to(end_b2, (32,))]),
    ])

    outs, oute = _build_sc_combine()(tab, locs, loce, params)
    return outs.reshape(BS, N), oute.reshape(BS, N)
